# baseline (speedup 1.0000x reference)
import jax
import jax.numpy as jnp
from jax import lax
from jax.experimental import pallas as pl
from jax.experimental.pallas import tpu as pltpu

_C = 21
_TOPK = 400
_NMS_T = 0.45
_ARM_VAR = (0.1, 0.2)
_ODM_VAR = (0.1, 0.2)
_POS_T = 0.01
_CONF_T = 0.01
_LANES = 128


def _sm(x):
    m = x.max(axis=-1, keepdims=True)
    e = jnp.exp(x - m)
    return e / e.sum(axis=-1, keepdims=True)


def _nms_body(score_ref, x1_ref, y1_ref, x2_ref, y2_ref, pidx_ref,
              outs_ref, ox1_ref, oy1_ref, ox2_ref, oy2_ref,
              act_ref, area_ref):
    score0 = score_ref[...]
    x1s = x1_ref[...]
    y1s = y1_ref[...]
    x2s = x2_ref[...]
    y2s = y2_ref[...]
    act_ref[...] = jnp.where(score0 > 0.0, 1.0, 0.0)
    area_ref[...] = (x2s - x1s) * (y2s - y1s)
    zeros = jnp.zeros_like(score0)
    outs_ref[...] = zeros
    ox1_ref[...] = zeros
    oy1_ref[...] = zeros
    ox2_ref[...] = zeros
    oy2_ref[...] = zeros

    def body(t, carry):
        act = act_ref[...] > 0.5
        score = score_ref[...]
        x1 = x1_ref[...]
        y1 = y1_ref[...]
        x2 = x2_ref[...]
        y2 = y2_ref[...]
        area = area_ref[...]
        pidx = pidx_ref[...]
        ms = jnp.where(act, score, -1.0)
        m = jnp.max(ms, axis=0, keepdims=True)
        has = m > 0.0
        cand = act & (score == m)
        tie = jnp.where(cand, pidx, -1.0)
        pmax = jnp.max(tie, axis=0, keepdims=True)
        onehot = cand & (pidx == pmax)
        oh = jnp.where(onehot, 1.0, 0.0)
        px1 = jnp.sum(oh * x1, axis=0, keepdims=True)
        py1 = jnp.sum(oh * y1, axis=0, keepdims=True)
        px2 = jnp.sum(oh * x2, axis=0, keepdims=True)
        py2 = jnp.sum(oh * y2, axis=0, keepdims=True)
        parea = jnp.sum(oh * area, axis=0, keepdims=True)
        xx1 = jnp.maximum(x1, px1)
        yy1 = jnp.maximum(y1, py1)
        xx2 = jnp.minimum(x2, px2)
        yy2 = jnp.minimum(y2, py2)
        w = jnp.clip(xx2 - xx1, 0.0, None)
        h = jnp.clip(yy2 - yy1, 0.0, None)
        inter = w * h
        union = (area - inter) + parea
        iou = inter / union
        keep = act & (iou <= _NMS_T) & jnp.logical_not(onehot)
        act_ref[...] = jnp.where(keep, 1.0, 0.0)
        outs_ref[pl.ds(t, 1), :] = jnp.where(has, m, 0.0)
        ox1_ref[pl.ds(t, 1), :] = jnp.where(has, px1, 0.0)
        oy1_ref[pl.ds(t, 1), :] = jnp.where(has, py1, 0.0)
        ox2_ref[pl.ds(t, 1), :] = jnp.where(has, px2, 0.0)
        oy2_ref[pl.ds(t, 1), :] = jnp.where(has, py2, 0.0)
        return carry

    lax.fori_loop(0, _TOPK, body, 0)


def _thresh_body(masked_ref, tau_ref, jcut_ref):
    v = masked_ref[...]
    kb = jax.lax.bitcast_convert_type(v, jnp.uint32)
    key = jnp.where(v < 0.0, ~kb, kb | jnp.uint32(0x80000000))
    rows = v.shape[0]
    idx = lax.broadcasted_iota(jnp.int32, v.shape, 1)

    def cnt_ge(cand):
        return jnp.sum((key >= cand).astype(jnp.float32), axis=1, keepdims=True)

    def bit_body(i, cur):
        bit = 31 - i
        cand = cur | (jnp.uint32(1) << bit)
        return jnp.where(cnt_ge(cand) >= float(_TOPK), cand, cur)

    cur = lax.fori_loop(0, 32, bit_body, jnp.zeros((rows, 1), jnp.uint32))

    n_gt = jnp.sum((key > cur).astype(jnp.float32), axis=1, keepdims=True)
    r = float(_TOPK) - n_gt
    is_tie = key == cur

    def jbit_body(i, jcur):
        bit = 14 - i
        jcand = jcur | (jnp.int32(1) << bit)
        cnt = jnp.sum((is_tie & (idx >= jcand)).astype(jnp.float32), axis=1,
                      keepdims=True)
        return jnp.where(cnt >= r, jcand, jcur)

    jcur = lax.fori_loop(0, 15, jbit_body, jnp.zeros((rows, 1), jnp.int32))

    tau_bits = jnp.where((cur >> 31) > 0, cur ^ jnp.uint32(0x80000000), ~cur)
    tau = jax.lax.bitcast_convert_type(tau_bits, jnp.float32)
    tau_ref[...] = jnp.broadcast_to(tau, (rows, 16))
    jcut_ref[...] = jnp.broadcast_to(jcur, (rows, 16))


def _run_thresh(masked_pad):
    rows = masked_pad.shape[0]
    return pl.pallas_call(
        _thresh_body,
        out_shape=[jax.ShapeDtypeStruct((rows, 16), jnp.float32),
                   jax.ShapeDtypeStruct((rows, 16), jnp.int32)],
    )(masked_pad)


def _run_nms(score_t, x1_t, y1_t, x2_t, y2_t, pidx_t):
    shp = jax.ShapeDtypeStruct((_TOPK, _LANES), jnp.float32)
    return pl.pallas_call(
        _nms_body,
        out_shape=[shp] * 5,
        scratch_shapes=[pltpu.VMEM((_TOPK, _LANES), jnp.float32)] * 2,
    )(score_t, x1_t, y1_t, x2_t, y2_t, pidx_t)


def kernel(arm_loc_data, arm_conf_data, odm_loc_data, odm_conf_data, prior_data):
    num, P, _ = arm_loc_data.shape
    nc = _C - 1
    npb = num * nc

    arm_score = _sm(arm_conf_data)
    score = _sm(odm_conf_data)
    centers = prior_data[None, :, :2] + arm_loc_data[:, :, :2] * _ARM_VAR[0] * prior_data[None, :, 2:]
    wh = prior_data[None, :, 2:] * jnp.exp(arm_loc_data[:, :, 2:] * _ARM_VAR[1])
    refined = jnp.concatenate([centers, wh], axis=2)
    xy = refined[..., :2] + odm_loc_data[..., :2] * _ODM_VAR[0] * refined[..., 2:]
    bwh = refined[..., 2:] * jnp.exp(odm_loc_data[..., 2:] * _ODM_VAR[1])
    x1y1 = xy - bwh / 2.0
    x2y2 = bwh + x1y1
    all_boxes = jnp.concatenate([x1y1, x2y2], axis=-1)

    flag = arm_score[:, :, 1] > _POS_T
    cls_scores = jnp.transpose(score, (0, 2, 1))[:, 1:, :]
    valid = flag[:, None, :] & (cls_scores > _CONF_T)
    masked = jnp.where(valid, cls_scores, -1.0).reshape(npb, P)

    masked_pad = jnp.pad(masked, ((0, 0), (0, 16384 - P)), constant_values=-2.0)
    tau16, jcut16 = _run_thresh(masked_pad)
    tau = tau16[:, :1]
    jcut = jcut16[:, :1]

    pidx_row = jnp.arange(P, dtype=jnp.int32)[None, :]
    sel = (masked > tau) | ((masked == tau) & (pidx_row >= jcut))
    dst = jnp.where(sel, jnp.cumsum(sel.astype(jnp.int32), axis=1) - 1, _TOPK)
    rows = jnp.arange(npb, dtype=jnp.int32)[:, None]
    vals = jnp.full((npb, _TOPK + 1), -1.0, jnp.float32).at[rows, dst].set(
        masked, mode='drop')[:, :_TOPK]
    idxs = jnp.zeros((npb, _TOPK + 1), jnp.int32).at[rows, dst].set(
        jnp.broadcast_to(pidx_row, masked.shape), mode='drop')[:, :_TOPK]
    bidx = (jnp.arange(npb) // nc)[:, None]
    cboxes = all_boxes[bidx, idxs]
    pidx = idxs.astype(jnp.float32)

    def plane(a, pad):
        a = jnp.pad(a, ((0, _LANES - npb), (0, 0)), constant_values=pad)
        return a.T

    score_t = plane(vals, -1.0)
    x1_t = plane(cboxes[..., 0], 0.0)
    y1_t = plane(cboxes[..., 1], 0.0)
    x2_t = plane(cboxes[..., 2], 0.0)
    y2_t = plane(cboxes[..., 3], 0.0)
    pidx_t = plane(pidx, 0.0)

    outs, ox1, oy1, ox2, oy2 = _run_nms(score_t, x1_t, y1_t, x2_t, y2_t, pidx_t)

    sel_s = outs.T[:npb]
    dets = jnp.stack([sel_s, ox1.T[:npb], oy1.T[:npb], ox2.T[:npb], oy2.T[:npb]], axis=-1)
    dets = dets.reshape(num, nc, _TOPK, 5)
    output = jnp.zeros((num, _C, _TOPK, 5), dtype=jnp.float32)
    output = output.at[:, 1:].set(dets)
    return output


# speedup vs baseline: 8.4510x; 8.4510x over previous
import jax
import jax.numpy as jnp
from jax import lax
from jax.experimental import pallas as pl
from jax.experimental.pallas import tpu as pltpu

_C = 21
_TOPK = 400
_NMS_T = 0.45
_ARM_VAR = (0.1, 0.2)
_ODM_VAR = (0.1, 0.2)
_POS_T = 0.01
_CONF_T = 0.01
_LANES = 128


def _sm(x):
    m = x.max(axis=-1, keepdims=True)
    e = jnp.exp(x - m)
    return e / e.sum(axis=-1, keepdims=True)


def _nms_body(score_ref, x1_ref, y1_ref, x2_ref, y2_ref, pidx_ref,
              outs_ref, ox1_ref, oy1_ref, ox2_ref, oy2_ref,
              act_ref, area_ref):
    score0 = score_ref[...]
    x1s = x1_ref[...]
    y1s = y1_ref[...]
    x2s = x2_ref[...]
    y2s = y2_ref[...]
    act_ref[...] = jnp.where(score0 > 0.0, 1.0, 0.0)
    area_ref[...] = (x2s - x1s) * (y2s - y1s)
    zeros = jnp.zeros_like(score0)
    outs_ref[...] = zeros
    ox1_ref[...] = zeros
    oy1_ref[...] = zeros
    ox2_ref[...] = zeros
    oy2_ref[...] = zeros

    def body(t, carry):
        act = act_ref[...] > 0.5
        score = score_ref[...]
        x1 = x1_ref[...]
        y1 = y1_ref[...]
        x2 = x2_ref[...]
        y2 = y2_ref[...]
        area = area_ref[...]
        pidx = pidx_ref[...]
        ms = jnp.where(act, score, -1.0)
        m = jnp.max(ms, axis=0, keepdims=True)
        has = m > 0.0
        cand = act & (score == m)
        tie = jnp.where(cand, pidx, -1.0)
        pmax = jnp.max(tie, axis=0, keepdims=True)
        onehot = cand & (pidx == pmax)
        oh = jnp.where(onehot, 1.0, 0.0)
        px1 = jnp.sum(oh * x1, axis=0, keepdims=True)
        py1 = jnp.sum(oh * y1, axis=0, keepdims=True)
        px2 = jnp.sum(oh * x2, axis=0, keepdims=True)
        py2 = jnp.sum(oh * y2, axis=0, keepdims=True)
        parea = jnp.sum(oh * area, axis=0, keepdims=True)
        xx1 = jnp.maximum(x1, px1)
        yy1 = jnp.maximum(y1, py1)
        xx2 = jnp.minimum(x2, px2)
        yy2 = jnp.minimum(y2, py2)
        w = jnp.clip(xx2 - xx1, 0.0, None)
        h = jnp.clip(yy2 - yy1, 0.0, None)
        inter = w * h
        union = (area - inter) + parea
        iou = inter / union
        keep = act & (iou <= _NMS_T) & jnp.logical_not(onehot)
        act_ref[...] = jnp.where(keep, 1.0, 0.0)
        outs_ref[pl.ds(t, 1), :] = jnp.where(has, m, 0.0)
        ox1_ref[pl.ds(t, 1), :] = jnp.where(has, px1, 0.0)
        oy1_ref[pl.ds(t, 1), :] = jnp.where(has, py1, 0.0)
        ox2_ref[pl.ds(t, 1), :] = jnp.where(has, px2, 0.0)
        oy2_ref[pl.ds(t, 1), :] = jnp.where(has, py2, 0.0)
        return carry

    lax.fori_loop(0, _TOPK, body, 0)


def _thresh_body(masked_ref, tau_ref, jcut_ref):
    v = masked_ref[...]
    kb = jax.lax.bitcast_convert_type(v, jnp.uint32)
    key = jnp.where(v < 0.0, ~kb, kb | jnp.uint32(0x80000000))
    rows = v.shape[0]
    idx = lax.broadcasted_iota(jnp.int32, v.shape, 1)

    def cnt_ge(cand):
        return jnp.sum((key >= cand).astype(jnp.float32), axis=1, keepdims=True)

    def bit_body(i, cur):
        bit = 31 - i
        cand = cur | (jnp.uint32(1) << bit)
        return jnp.where(cnt_ge(cand) >= float(_TOPK), cand, cur)

    cur = lax.fori_loop(0, 32, bit_body, jnp.zeros((rows, 1), jnp.uint32))

    n_gt = jnp.sum((key > cur).astype(jnp.float32), axis=1, keepdims=True)
    r = float(_TOPK) - n_gt
    is_tie = key == cur

    def jbit_body(i, jcur):
        bit = 14 - i
        jcand = jcur | (jnp.int32(1) << bit)
        cnt = jnp.sum((is_tie & (idx >= jcand)).astype(jnp.float32), axis=1,
                      keepdims=True)
        return jnp.where(cnt >= r, jcand, jcur)

    jcur = lax.fori_loop(0, 15, jbit_body, jnp.zeros((rows, 1), jnp.int32))

    tau_bits = jnp.where((cur >> 31) > 0, cur ^ jnp.uint32(0x80000000), ~cur)
    tau = jax.lax.bitcast_convert_type(tau_bits, jnp.float32)
    tau_ref[...] = jnp.broadcast_to(tau, (rows, 16))
    jcut_ref[...] = jnp.broadcast_to(jcur, (rows, 16))


def _run_thresh(masked_pad):
    rows = masked_pad.shape[0]
    return pl.pallas_call(
        _thresh_body,
        out_shape=[jax.ShapeDtypeStruct((rows, 16), jnp.float32),
                   jax.ShapeDtypeStruct((rows, 16), jnp.int32)],
    )(masked_pad)


def _run_nms(score_t, x1_t, y1_t, x2_t, y2_t, pidx_t):
    shp = jax.ShapeDtypeStruct((_TOPK, _LANES), jnp.float32)
    return pl.pallas_call(
        _nms_body,
        out_shape=[shp] * 5,
        scratch_shapes=[pltpu.VMEM((_TOPK, _LANES), jnp.float32)] * 2,
    )(score_t, x1_t, y1_t, x2_t, y2_t, pidx_t)


def kernel(arm_loc_data, arm_conf_data, odm_loc_data, odm_conf_data, prior_data):
    num, P, _ = arm_loc_data.shape
    nc = _C - 1
    npb = num * nc

    arm_score = _sm(arm_conf_data)
    score = _sm(odm_conf_data)
    centers = prior_data[None, :, :2] + arm_loc_data[:, :, :2] * _ARM_VAR[0] * prior_data[None, :, 2:]
    wh = prior_data[None, :, 2:] * jnp.exp(arm_loc_data[:, :, 2:] * _ARM_VAR[1])
    refined = jnp.concatenate([centers, wh], axis=2)
    xy = refined[..., :2] + odm_loc_data[..., :2] * _ODM_VAR[0] * refined[..., 2:]
    bwh = refined[..., 2:] * jnp.exp(odm_loc_data[..., 2:] * _ODM_VAR[1])
    x1y1 = xy - bwh / 2.0
    x2y2 = bwh + x1y1
    all_boxes = jnp.concatenate([x1y1, x2y2], axis=-1)

    flag = arm_score[:, :, 1] > _POS_T
    cls_scores = jnp.transpose(score, (0, 2, 1))[:, 1:, :]
    valid = flag[:, None, :] & (cls_scores > _CONF_T)
    masked = jnp.where(valid, cls_scores, -1.0).reshape(npb, P)

    masked_pad = jnp.pad(masked, ((0, 0), (0, 16384 - P)), constant_values=-2.0)
    tau16, jcut16 = _run_thresh(masked_pad)
    tau = tau16[:, :1]
    jcut = jcut16[:, :1]

    pidx_row = jnp.arange(P, dtype=jnp.int32)[None, :]
    sel = (masked > tau) | ((masked == tau) & (pidx_row >= jcut))
    keys = jnp.where(sel, masked, -2.0)
    vals, idxs = lax.top_k(keys, _TOPK)
    bidx = (jnp.arange(npb) // nc)[:, None]
    cboxes = all_boxes[bidx, idxs]
    pidx = idxs.astype(jnp.float32)

    def plane(a, pad):
        a = jnp.pad(a, ((0, _LANES - npb), (0, 0)), constant_values=pad)
        return a.T

    score_t = plane(vals, -1.0)
    x1_t = plane(cboxes[..., 0], 0.0)
    y1_t = plane(cboxes[..., 1], 0.0)
    x2_t = plane(cboxes[..., 2], 0.0)
    y2_t = plane(cboxes[..., 3], 0.0)
    pidx_t = plane(pidx, 0.0)

    outs, ox1, oy1, ox2, oy2 = _run_nms(score_t, x1_t, y1_t, x2_t, y2_t, pidx_t)

    sel_s = outs.T[:npb]
    dets = jnp.stack([sel_s, ox1.T[:npb], oy1.T[:npb], ox2.T[:npb], oy2.T[:npb]], axis=-1)
    dets = dets.reshape(num, nc, _TOPK, 5)
    output = jnp.zeros((num, _C, _TOPK, 5), dtype=jnp.float32)
    output = output.at[:, 1:].set(dets)
    return output


# speedup vs baseline: 14.8184x; 1.7534x over previous
import jax
import jax.numpy as jnp
from jax import lax
from jax.experimental import pallas as pl
from jax.experimental.pallas import tpu as pltpu

_C = 21
_TOPK = 400
_NMS_T = 0.45
_ARM_VAR = (0.1, 0.2)
_ODM_VAR = (0.1, 0.2)
_POS_T = 0.01
_CONF_T = 0.01
_LANES = 128


def _sm(x):
    m = x.max(axis=-1, keepdims=True)
    e = jnp.exp(x - m)
    return e / e.sum(axis=-1, keepdims=True)


def _nms_body(score_ref, x1_ref, y1_ref, x2_ref, y2_ref, pidx_ref,
              outs_ref, ox1_ref, oy1_ref, ox2_ref, oy2_ref,
              act_ref, area_ref):
    score0 = score_ref[...]
    x1s = x1_ref[...]
    y1s = y1_ref[...]
    x2s = x2_ref[...]
    y2s = y2_ref[...]
    act_ref[...] = jnp.where(score0 > 0.0, 1.0, 0.0)
    area_ref[...] = (x2s - x1s) * (y2s - y1s)
    zeros = jnp.zeros_like(score0)
    outs_ref[...] = zeros
    ox1_ref[...] = zeros
    oy1_ref[...] = zeros
    ox2_ref[...] = zeros
    oy2_ref[...] = zeros

    def body(t, carry):
        act = act_ref[...] > 0.5
        score = score_ref[...]
        x1 = x1_ref[...]
        y1 = y1_ref[...]
        x2 = x2_ref[...]
        y2 = y2_ref[...]
        area = area_ref[...]
        pidx = pidx_ref[...]
        ms = jnp.where(act, score, -1.0)
        m = jnp.max(ms, axis=0, keepdims=True)
        has = m > 0.0
        cand = act & (score == m)
        tie = jnp.where(cand, pidx, -1.0)
        pmax = jnp.max(tie, axis=0, keepdims=True)
        onehot = cand & (pidx == pmax)
        oh = jnp.where(onehot, 1.0, 0.0)
        px1 = jnp.sum(oh * x1, axis=0, keepdims=True)
        py1 = jnp.sum(oh * y1, axis=0, keepdims=True)
        px2 = jnp.sum(oh * x2, axis=0, keepdims=True)
        py2 = jnp.sum(oh * y2, axis=0, keepdims=True)
        parea = jnp.sum(oh * area, axis=0, keepdims=True)
        xx1 = jnp.maximum(x1, px1)
        yy1 = jnp.maximum(y1, py1)
        xx2 = jnp.minimum(x2, px2)
        yy2 = jnp.minimum(y2, py2)
        w = jnp.clip(xx2 - xx1, 0.0, None)
        h = jnp.clip(yy2 - yy1, 0.0, None)
        inter = w * h
        union = (area - inter) + parea
        iou = inter / union
        keep = act & (iou <= _NMS_T) & jnp.logical_not(onehot)
        act_ref[...] = jnp.where(keep, 1.0, 0.0)
        outs_ref[pl.ds(t, 1), :] = jnp.where(has, m, 0.0)
        ox1_ref[pl.ds(t, 1), :] = jnp.where(has, px1, 0.0)
        oy1_ref[pl.ds(t, 1), :] = jnp.where(has, py1, 0.0)
        ox2_ref[pl.ds(t, 1), :] = jnp.where(has, px2, 0.0)
        oy2_ref[pl.ds(t, 1), :] = jnp.where(has, py2, 0.0)
        return carry

    lax.fori_loop(0, _TOPK, body, 0)


def _thresh_body(masked_ref, tau_ref, jcut_ref):
    v = masked_ref[...]
    kb = jax.lax.bitcast_convert_type(v, jnp.uint32)
    key = jnp.where(v < 0.0, ~kb, kb | jnp.uint32(0x80000000))
    rows = v.shape[0]
    idx = lax.broadcasted_iota(jnp.int32, v.shape, 1)

    def cnt_ge(cand):
        return jnp.sum((key >= cand).astype(jnp.float32), axis=1, keepdims=True)

    def bit_body(i, cur):
        bit = 31 - i
        cand = cur | (jnp.uint32(1) << bit)
        return jnp.where(cnt_ge(cand) >= float(_TOPK), cand, cur)

    cur = lax.fori_loop(0, 32, bit_body, jnp.zeros((rows, 1), jnp.uint32))

    n_gt = jnp.sum((key > cur).astype(jnp.float32), axis=1, keepdims=True)
    r = float(_TOPK) - n_gt
    is_tie = key == cur

    def jbit_body(i, jcur):
        bit = 14 - i
        jcand = jcur | (jnp.int32(1) << bit)
        cnt = jnp.sum((is_tie & (idx >= jcand)).astype(jnp.float32), axis=1,
                      keepdims=True)
        return jnp.where(cnt >= r, jcand, jcur)

    jcur = lax.fori_loop(0, 15, jbit_body, jnp.zeros((rows, 1), jnp.int32))

    tau_bits = jnp.where((cur >> 31) > 0, cur ^ jnp.uint32(0x80000000), ~cur)
    tau = jax.lax.bitcast_convert_type(tau_bits, jnp.float32)
    tau_ref[...] = jnp.broadcast_to(tau, (rows, 16))
    jcut_ref[...] = jnp.broadcast_to(jcur, (rows, 16))


def _run_thresh(masked_pad):
    rows = masked_pad.shape[0]
    return pl.pallas_call(
        _thresh_body,
        out_shape=[jax.ShapeDtypeStruct((rows, 16), jnp.float32),
                   jax.ShapeDtypeStruct((rows, 16), jnp.int32)],
    )(masked_pad)


def _run_nms(score_t, x1_t, y1_t, x2_t, y2_t, pidx_t):
    shp = jax.ShapeDtypeStruct((_TOPK, _LANES), jnp.float32)
    return pl.pallas_call(
        _nms_body,
        out_shape=[shp] * 5,
        scratch_shapes=[pltpu.VMEM((_TOPK, _LANES), jnp.float32)] * 2,
    )(score_t, x1_t, y1_t, x2_t, y2_t, pidx_t)


def kernel(arm_loc_data, arm_conf_data, odm_loc_data, odm_conf_data, prior_data):
    num, P, _ = arm_loc_data.shape
    nc = _C - 1
    npb = num * nc

    arm_score = _sm(arm_conf_data)
    score = _sm(odm_conf_data)
    centers = prior_data[None, :, :2] + arm_loc_data[:, :, :2] * _ARM_VAR[0] * prior_data[None, :, 2:]
    wh = prior_data[None, :, 2:] * jnp.exp(arm_loc_data[:, :, 2:] * _ARM_VAR[1])
    refined = jnp.concatenate([centers, wh], axis=2)
    xy = refined[..., :2] + odm_loc_data[..., :2] * _ODM_VAR[0] * refined[..., 2:]
    bwh = refined[..., 2:] * jnp.exp(odm_loc_data[..., 2:] * _ODM_VAR[1])
    x1y1 = xy - bwh / 2.0
    x2y2 = bwh + x1y1
    all_boxes = jnp.concatenate([x1y1, x2y2], axis=-1)

    flag = arm_score[:, :, 1] > _POS_T
    cls_scores = jnp.transpose(score, (0, 2, 1))[:, 1:, :]
    valid = flag[:, None, :] & (cls_scores > _CONF_T)
    masked = jnp.where(valid, cls_scores, -1.0).reshape(npb, P)

    masked_pad = jnp.pad(masked, ((0, 0), (0, 16384 - P)), constant_values=-2.0)
    tau16, jcut16 = _run_thresh(masked_pad)
    tau = tau16[:, :1]
    jcut = jcut16[:, :1]

    pidx_row = jnp.arange(16384, dtype=jnp.int32)[None, :]
    sel = (masked_pad > tau) | ((masked_pad == tau) & (pidx_row >= jcut))
    sel3 = sel.reshape(npb, 128, 128)
    lane = jnp.arange(128, dtype=jnp.int32)
    key = jnp.where(sel3, lane[None, None, :], 1000 + lane[None, None, :])
    sk = jnp.sort(key, axis=2)
    bc = sel3.sum(axis=2, dtype=jnp.int32)
    boff = jnp.concatenate([jnp.zeros((npb, 1), jnp.int32),
                            jnp.cumsum(bc, axis=1)[:, :-1]], axis=1)
    s_ids = jnp.arange(_TOPK, dtype=jnp.int32)
    bos = (boff[:, None, :] <= s_ids[None, :, None]).sum(axis=2).astype(jnp.int32) - 1
    roff = s_ids[None, :] - jnp.take_along_axis(boff, bos, axis=1)
    lanes = jnp.take_along_axis(sk.reshape(npb, 16384), bos * 128 + roff, axis=1)
    idxs = bos * 128 + (lanes % 1000)
    vals = jnp.take_along_axis(masked, idxs, axis=1)
    bidx = (jnp.arange(npb) // nc)[:, None]
    cboxes = all_boxes[bidx, idxs]
    pidx = idxs.astype(jnp.float32)

    def plane(a, pad):
        a = jnp.pad(a, ((0, _LANES - npb), (0, 0)), constant_values=pad)
        return a.T

    score_t = plane(vals, -1.0)
    x1_t = plane(cboxes[..., 0], 0.0)
    y1_t = plane(cboxes[..., 1], 0.0)
    x2_t = plane(cboxes[..., 2], 0.0)
    y2_t = plane(cboxes[..., 3], 0.0)
    pidx_t = plane(pidx, 0.0)

    outs, ox1, oy1, ox2, oy2 = _run_nms(score_t, x1_t, y1_t, x2_t, y2_t, pidx_t)

    sel_s = outs.T[:npb]
    dets = jnp.stack([sel_s, ox1.T[:npb], oy1.T[:npb], ox2.T[:npb], oy2.T[:npb]], axis=-1)
    dets = dets.reshape(num, nc, _TOPK, 5)
    output = jnp.zeros((num, _C, _TOPK, 5), dtype=jnp.float32)
    output = output.at[:, 1:].set(dets)
    return output


# speedup vs baseline: 19.6028x; 1.3229x over previous
import jax
import jax.numpy as jnp
from jax import lax
from jax.experimental import pallas as pl
from jax.experimental.pallas import tpu as pltpu

_C = 21
_TOPK = 400
_NMS_T = 0.45
_ARM_VAR = (0.1, 0.2)
_ODM_VAR = (0.1, 0.2)
_POS_T = 0.01
_CONF_T = 0.01
_LANES = 128


def _sm(x):
    m = x.max(axis=-1, keepdims=True)
    e = jnp.exp(x - m)
    return e / e.sum(axis=-1, keepdims=True)


def _nms_body(score_ref, x1_ref, y1_ref, x2_ref, y2_ref, pidx_ref,
              outs_ref, ox1_ref, oy1_ref, ox2_ref, oy2_ref,
              act_ref, area_ref):
    score0 = score_ref[...]
    x1s = x1_ref[...]
    y1s = y1_ref[...]
    x2s = x2_ref[...]
    y2s = y2_ref[...]
    act_ref[...] = jnp.where(score0 > 0.0, 1.0, 0.0)
    area_ref[...] = (x2s - x1s) * (y2s - y1s)
    zeros = jnp.zeros_like(score0)
    outs_ref[...] = zeros
    ox1_ref[...] = zeros
    oy1_ref[...] = zeros
    ox2_ref[...] = zeros
    oy2_ref[...] = zeros

    def body(t, carry):
        act = act_ref[...] > 0.5
        score = score_ref[...]
        x1 = x1_ref[...]
        y1 = y1_ref[...]
        x2 = x2_ref[...]
        y2 = y2_ref[...]
        area = area_ref[...]
        pidx = pidx_ref[...]
        ms = jnp.where(act, score, -1.0)
        m = jnp.max(ms, axis=0, keepdims=True)
        has = m > 0.0
        cand = act & (score == m)
        tie = jnp.where(cand, pidx, -1.0)
        pmax = jnp.max(tie, axis=0, keepdims=True)
        onehot = cand & (pidx == pmax)
        oh = jnp.where(onehot, 1.0, 0.0)
        px1 = jnp.sum(oh * x1, axis=0, keepdims=True)
        py1 = jnp.sum(oh * y1, axis=0, keepdims=True)
        px2 = jnp.sum(oh * x2, axis=0, keepdims=True)
        py2 = jnp.sum(oh * y2, axis=0, keepdims=True)
        parea = jnp.sum(oh * area, axis=0, keepdims=True)
        xx1 = jnp.maximum(x1, px1)
        yy1 = jnp.maximum(y1, py1)
        xx2 = jnp.minimum(x2, px2)
        yy2 = jnp.minimum(y2, py2)
        w = jnp.clip(xx2 - xx1, 0.0, None)
        h = jnp.clip(yy2 - yy1, 0.0, None)
        inter = w * h
        union = (area - inter) + parea
        iou = inter / union
        keep = act & (iou <= _NMS_T) & jnp.logical_not(onehot)
        act_ref[...] = jnp.where(keep, 1.0, 0.0)
        outs_ref[pl.ds(t, 1), :] = jnp.where(has, m, 0.0)
        ox1_ref[pl.ds(t, 1), :] = jnp.where(has, px1, 0.0)
        oy1_ref[pl.ds(t, 1), :] = jnp.where(has, py1, 0.0)
        ox2_ref[pl.ds(t, 1), :] = jnp.where(has, px2, 0.0)
        oy2_ref[pl.ds(t, 1), :] = jnp.where(has, py2, 0.0)
        return carry

    lax.fori_loop(0, _TOPK, body, 0)


def _thresh_body(masked_ref, tau_ref, jcut_ref):
    v = masked_ref[...]
    kb = jax.lax.bitcast_convert_type(v, jnp.uint32)
    key = jnp.where(v < 0.0, ~kb, kb | jnp.uint32(0x80000000))
    rows = v.shape[0]
    idx = lax.broadcasted_iota(jnp.int32, v.shape, 1)

    def cnt_ge(cand):
        return jnp.sum((key >= cand).astype(jnp.float32), axis=1, keepdims=True)

    def bit_body(i, cur):
        bit = 31 - i
        cand = cur | (jnp.uint32(1) << bit)
        return jnp.where(cnt_ge(cand) >= float(_TOPK), cand, cur)

    cur = lax.fori_loop(0, 32, bit_body, jnp.zeros((rows, 1), jnp.uint32))

    n_gt = jnp.sum((key > cur).astype(jnp.float32), axis=1, keepdims=True)
    r = float(_TOPK) - n_gt
    is_tie = key == cur

    def jbit_body(i, jcur):
        bit = 14 - i
        jcand = jcur | (jnp.int32(1) << bit)
        cnt = jnp.sum((is_tie & (idx >= jcand)).astype(jnp.float32), axis=1,
                      keepdims=True)
        return jnp.where(cnt >= r, jcand, jcur)

    jcur = lax.fori_loop(0, 15, jbit_body, jnp.zeros((rows, 1), jnp.int32))

    tau_bits = jnp.where((cur >> 31) > 0, cur ^ jnp.uint32(0x80000000), ~cur)
    tau = jax.lax.bitcast_convert_type(tau_bits, jnp.float32)
    tau_ref[...] = jnp.broadcast_to(tau, (rows, 16))
    jcut_ref[...] = jnp.broadcast_to(jcur, (rows, 16))


def _run_thresh(masked_pad):
    rows = masked_pad.shape[0]
    return pl.pallas_call(
        _thresh_body,
        out_shape=[jax.ShapeDtypeStruct((rows, 16), jnp.float32),
                   jax.ShapeDtypeStruct((rows, 16), jnp.int32)],
    )(masked_pad)


def _run_nms(score_t, x1_t, y1_t, x2_t, y2_t, pidx_t):
    shp = jax.ShapeDtypeStruct((_TOPK, _LANES), jnp.float32)
    return pl.pallas_call(
        _nms_body,
        out_shape=[shp] * 5,
        scratch_shapes=[pltpu.VMEM((_TOPK, _LANES), jnp.float32)] * 2,
    )(score_t, x1_t, y1_t, x2_t, y2_t, pidx_t)


def kernel(arm_loc_data, arm_conf_data, odm_loc_data, odm_conf_data, prior_data):
    num, P, _ = arm_loc_data.shape
    nc = _C - 1
    npb = num * nc

    arm_score = _sm(arm_conf_data)
    score = _sm(odm_conf_data)
    centers = prior_data[None, :, :2] + arm_loc_data[:, :, :2] * _ARM_VAR[0] * prior_data[None, :, 2:]
    wh = prior_data[None, :, 2:] * jnp.exp(arm_loc_data[:, :, 2:] * _ARM_VAR[1])
    refined = jnp.concatenate([centers, wh], axis=2)
    xy = refined[..., :2] + odm_loc_data[..., :2] * _ODM_VAR[0] * refined[..., 2:]
    bwh = refined[..., 2:] * jnp.exp(odm_loc_data[..., 2:] * _ODM_VAR[1])
    x1y1 = xy - bwh / 2.0
    x2y2 = bwh + x1y1
    all_boxes = jnp.concatenate([x1y1, x2y2], axis=-1)

    flag = arm_score[:, :, 1] > _POS_T
    cls_scores = jnp.transpose(score, (0, 2, 1))[:, 1:, :]
    valid = flag[:, None, :] & (cls_scores > _CONF_T)
    masked = jnp.where(valid, cls_scores, -1.0).reshape(npb, P)

    masked_pad = jnp.pad(masked, ((0, 0), (0, 16384 - P)), constant_values=-2.0)
    tau16, jcut16 = _run_thresh(masked_pad)
    tau = tau16[:, :1]
    jcut = jcut16[:, :1]

    pidx_row = jnp.arange(16384, dtype=jnp.int32)[None, :]
    sel = (masked_pad > tau) | ((masked_pad == tau) & (pidx_row >= jcut))
    sel3 = sel.reshape(npb, 128, 128)
    lane16 = jnp.arange(128, dtype=jnp.uint16)
    key = jnp.where(sel3, lane16[None, None, :],
                    jnp.uint16(1000) + lane16[None, None, :])
    sk = jnp.sort(key, axis=2)
    bc = sel3.sum(axis=2, dtype=jnp.int32)
    boff = jnp.concatenate([jnp.zeros((npb, 1), jnp.int32),
                            jnp.cumsum(bc, axis=1)[:, :-1]], axis=1)
    s_ids = jnp.arange(_TOPK, dtype=jnp.int32)
    le = boff[:, None, :] <= s_ids[None, :, None]
    bos = le.sum(axis=2).astype(jnp.int32) - 1
    boff_at_bos = jnp.max(jnp.where(le, boff[:, None, :], -1), axis=2)
    roff = s_ids[None, :] - boff_at_bos
    lanes = jnp.take_along_axis(sk.reshape(npb, 16384), bos * 128 + roff,
                                axis=1).astype(jnp.int32)
    idxs = bos * 128 + (lanes % 1000)
    vals = jnp.take_along_axis(masked, idxs, axis=1)
    bidx = (jnp.arange(npb) // nc)[:, None]
    cboxes = all_boxes[bidx, idxs]
    pidx = idxs.astype(jnp.float32)

    def plane(a, pad):
        a = jnp.pad(a, ((0, _LANES - npb), (0, 0)), constant_values=pad)
        return a.T

    score_t = plane(vals, -1.0)
    x1_t = plane(cboxes[..., 0], 0.0)
    y1_t = plane(cboxes[..., 1], 0.0)
    x2_t = plane(cboxes[..., 2], 0.0)
    y2_t = plane(cboxes[..., 3], 0.0)
    pidx_t = plane(pidx, 0.0)

    outs, ox1, oy1, ox2, oy2 = _run_nms(score_t, x1_t, y1_t, x2_t, y2_t, pidx_t)

    sel_s = outs.T[:npb]
    dets = jnp.stack([sel_s, ox1.T[:npb], oy1.T[:npb], ox2.T[:npb], oy2.T[:npb]], axis=-1)
    dets = dets.reshape(num, nc, _TOPK, 5)
    output = jnp.zeros((num, _C, _TOPK, 5), dtype=jnp.float32)
    output = output.at[:, 1:].set(dets)
    return output
